# baseline (speedup 1.0000x reference)
import functools

import jax
import jax.numpy as jnp
from jax import lax
from jax.experimental import pallas as pl
from jax.experimental.pallas import tpu as pltpu
from jax.experimental.pallas import tpu_sc as plsc

NUM_CLASSES = 1000
ROWS = 4096 * 26
NUM_WORKERS = 32
ROWS_PER_W = ROWS // NUM_WORKERS
CHUNK_ROWS = 32
NUM_CHUNKS = ROWS_PER_W // CHUNK_ROWS
CHUNK_WORDS = CHUNK_ROWS * NUM_CLASSES
LANES = 16

_mesh = plsc.VectorSubcoreMesh(core_axis_name="c", subcore_axis_name="s")


@functools.partial(
    pl.kernel,
    out_type=jax.ShapeDtypeStruct((ROWS * NUM_CLASSES,), jnp.int32),
    mesh=_mesh,
    compiler_params=pltpu.CompilerParams(needs_layout_passes=False),
    scratch_types=[
        pltpu.VMEM((ROWS_PER_W,), jnp.int32),
        pltpu.VMEM((CHUNK_WORDS,), jnp.int32),
        pltpu.VMEM((CHUNK_WORDS,), jnp.int32),
        pltpu.SemaphoreType.DMA,
        pltpu.SemaphoreType.DMA,
    ],
)
def _onehot_sc(idx_hbm, out_hbm, idx_all, buf0, buf1, sem0, sem1):
    wid = lax.axis_index("s") * 2 + lax.axis_index("c")
    base_row = wid * ROWS_PER_W

    pltpu.sync_copy(idx_hbm.at[pl.ds(base_row, ROWS_PER_W)], idx_all)

    zeros16 = jnp.zeros((LANES,), jnp.int32)
    ones16 = jnp.ones((LANES,), jnp.int32)
    lane = lax.iota(jnp.int32, LANES)

    bufs = (buf0, buf1)
    sems = (sem0, sem1)

    def memset(i, carry):
        buf0[pl.ds(i * LANES, LANES)] = zeros16
        buf1[pl.ds(i * LANES, LANES)] = zeros16
        return carry

    lax.fori_loop(0, CHUNK_WORDS // LANES, memset, 0, unroll=8)

    def positions(g, j):
        idxv = idx_all[pl.ds(g * CHUNK_ROWS + j * LANES, LANES)]
        return (lane + j * LANES) * NUM_CLASSES + idxv

    def scatter(g, buf, val):
        for j in range(CHUNK_ROWS // LANES):
            plsc.store_scatter(buf, [positions(g, j)], val)

    def start_out(g, buf, sem):
        off = (base_row + g * CHUNK_ROWS) * NUM_CLASSES
        pltpu.async_copy(buf, out_hbm.at[pl.ds(off, CHUNK_WORDS)], sem)

    def wait_out(g, buf, sem):
        off = (base_row + g * CHUNK_ROWS) * NUM_CLASSES
        pltpu.make_async_copy(buf, out_hbm.at[pl.ds(off, CHUNK_WORDS)], sem).wait()

    for b in range(2):
        scatter(b, bufs[b], ones16)
        start_out(b, bufs[b], sems[b])

    def outer(go, carry):
        for b in range(2):
            g = go * 2 + b
            wait_out(g - 2, bufs[b], sems[b])
            scatter(g - 2, bufs[b], zeros16)
            scatter(g, bufs[b], ones16)
            start_out(g, bufs[b], sems[b])
        return carry

    lax.fori_loop(1, NUM_CHUNKS // 2, outer, 0)

    for b in range(2):
        wait_out(NUM_CHUNKS - 2 + b, bufs[b], sems[b])


def kernel(inputs):
    flat = _onehot_sc(inputs.reshape(-1))
    return flat.reshape(inputs.shape[0], inputs.shape[1], NUM_CLASSES)


# speedup vs baseline: 1.3812x; 1.3812x over previous
import functools

import jax
import jax.numpy as jnp
from jax.experimental import pallas as pl
from jax.experimental.pallas import tpu as pltpu

NUM_CLASSES = 1000
ROWS = 4096 * 26
BLOCK_ROWS = 1024


def _body(idx_ref, out_ref):
    idx = idx_ref[...]
    classes = jax.lax.broadcasted_iota(jnp.int32, (BLOCK_ROWS, NUM_CLASSES), 1)
    out_ref[...] = (idx == classes).astype(jnp.int32)


@jax.jit
def _onehot_tc(idx_flat):
    return pl.pallas_call(
        _body,
        grid=(ROWS // BLOCK_ROWS,),
        in_specs=[pl.BlockSpec((BLOCK_ROWS, 1), lambda i: (i, 0))],
        out_specs=pl.BlockSpec((BLOCK_ROWS, NUM_CLASSES), lambda i: (i, 0)),
        out_shape=jax.ShapeDtypeStruct((ROWS, NUM_CLASSES), jnp.int32),
    )(idx_flat)


def kernel(inputs):
    flat = _onehot_tc(inputs.reshape(-1, 1))
    return flat.reshape(inputs.shape[0], inputs.shape[1], NUM_CLASSES)


# speedup vs baseline: 2.0176x; 1.4608x over previous
import jax
import jax.numpy as jnp
from jax.experimental import pallas as pl

NUM_CLASSES = 1000
NUM_EXAMPLES = 4096
NUM_FEATURES = 26
BLOCK_EXAMPLES = 128


def _body(idx_ref, out_ref):
    idx = idx_ref[...]
    classes = jax.lax.broadcasted_iota(
        jnp.int32, (BLOCK_EXAMPLES, NUM_FEATURES, NUM_CLASSES), 2)
    out_ref[...] = (idx[:, :, None] == classes).astype(jnp.int32)


@jax.jit
def _onehot_tc(idx):
    return pl.pallas_call(
        _body,
        grid=(NUM_EXAMPLES // BLOCK_EXAMPLES,),
        in_specs=[pl.BlockSpec((BLOCK_EXAMPLES, NUM_FEATURES), lambda i: (i, 0))],
        out_specs=pl.BlockSpec(
            (BLOCK_EXAMPLES, NUM_FEATURES, NUM_CLASSES), lambda i: (i, 0, 0)),
        out_shape=jax.ShapeDtypeStruct(
            (NUM_EXAMPLES, NUM_FEATURES, NUM_CLASSES), jnp.int32),
    )(idx)


def kernel(inputs):
    return _onehot_tc(inputs)


# speedup vs baseline: 2.0246x; 1.0035x over previous
import jax
import jax.numpy as jnp
from jax import lax
from jax.experimental import pallas as pl
from jax.experimental.pallas import tpu as pltpu

NUM_CLASSES = 1000
NUM_EXAMPLES = 4096
NUM_FEATURES = 26
CHUNK_EXAMPLES = 32
NUM_CHUNKS = NUM_EXAMPLES // CHUNK_EXAMPLES
NBUF = 8


def _body(idx_ref, out_ref, buf, sems):
    iota3 = jax.lax.broadcasted_iota(
        jnp.int32, (CHUNK_EXAMPLES, NUM_FEATURES, NUM_CLASSES), 2)

    def out_copy(c, b):
        return pltpu.make_async_copy(
            buf.at[b],
            out_ref.at[pl.ds(c * CHUNK_EXAMPLES, CHUNK_EXAMPLES)],
            sems.at[b],
        )

    def chunk(c, carry):
        b = lax.rem(c, NBUF)

        @pl.when(c >= NBUF)
        def _():
            out_copy(c - NBUF, b).wait()

        idx = idx_ref[pl.ds(c * CHUNK_EXAMPLES, CHUNK_EXAMPLES), :]
        buf[pl.ds(b, 1)] = ((idx[:, :, None] == iota3).astype(jnp.int32))[None]
        out_copy(c, b).start()
        return carry

    lax.fori_loop(0, NUM_CHUNKS, chunk, 0)

    def drain(c, carry):
        out_copy(c, lax.rem(c, NBUF)).wait()
        return carry

    lax.fori_loop(NUM_CHUNKS - NBUF, NUM_CHUNKS, drain, 0)


@jax.jit
def _onehot_tc(idx):
    return pl.pallas_call(
        _body,
        in_specs=[pl.BlockSpec(memory_space=pltpu.VMEM)],
        out_specs=pl.BlockSpec(memory_space=pl.ANY),
        out_shape=jax.ShapeDtypeStruct(
            (NUM_EXAMPLES, NUM_FEATURES, NUM_CLASSES), jnp.int32),
        scratch_shapes=[
            pltpu.VMEM((NBUF, CHUNK_EXAMPLES, NUM_FEATURES, NUM_CLASSES),
                       jnp.int32),
            pltpu.SemaphoreType.DMA((NBUF,)),
        ],
    )(idx)


def kernel(inputs):
    return _onehot_tc(inputs)


# speedup vs baseline: 9.5621x; 4.7229x over previous
import jax
import jax.numpy as jnp
from jax.experimental import pallas as pl

NUM_CLASSES = 1000
NUM_EXAMPLES = 4096
NUM_FEATURES = 26
BLOCK_F = 2
BLOCK_C = 200


def _body(idx_ref, out_ref):
    j = pl.program_id(1)
    idx = idx_ref[0]
    classes = jax.lax.broadcasted_iota(
        jnp.int32, (BLOCK_F, BLOCK_C, NUM_EXAMPLES), 1) + j * BLOCK_C
    out_ref[...] = (idx[:, None, :] == classes).astype(jnp.int32)


@jax.jit
def _onehot_t(idx_t):
    return pl.pallas_call(
        _body,
        grid=(NUM_FEATURES // BLOCK_F, NUM_CLASSES // BLOCK_C),
        in_specs=[pl.BlockSpec(
            (1, BLOCK_F, NUM_EXAMPLES), lambda i, j: (i, 0, 0))],
        out_specs=pl.BlockSpec(
            (BLOCK_F, BLOCK_C, NUM_EXAMPLES), lambda i, j: (i, j, 0)),
        out_shape=jax.ShapeDtypeStruct(
            (NUM_FEATURES, NUM_CLASSES, NUM_EXAMPLES), jnp.int32),
    )(idx_t)


def kernel(inputs):
    idx_t = inputs.T.reshape(NUM_FEATURES // BLOCK_F, BLOCK_F, NUM_EXAMPLES)
    out_t = _onehot_t(idx_t)
    return jnp.transpose(out_t, (2, 0, 1))
